# Initial kernel scaffold; baseline (speedup 1.0000x reference)
import jax
import jax.numpy as jnp
from jax.experimental import pallas as pl


def kernel(x, edge_index, edge_attr, batch, lin1, mW1_1, mb1_1, mW2_1, mb2_1, b1, lin2, mW1_2, mb1_2, mW2_2, mb2_2, b2, fcW, fcb):
    raise NotImplementedError("write your pallas kernel here")


# speedup vs baseline: 2.2467x; 2.2467x over previous
import functools

import jax
import jax.numpy as jnp
from jax import lax
from jax.experimental import pallas as pl
from jax.experimental.pallas import tpu as pltpu

N, E, D, DE, H = 10000, 320000, 128, 16, 64
EPAD = 323584
BE = 4096
BN = 1000


def _leaky(v):
    return jnp.where(v >= 0, v, 0.01 * v)


def _emlp_body(ea_ref, W1c_ref, b1c_ref, W2c_ref, b2c_ref, w1_ref, w2_ref):
    i = pl.program_id(0)
    h = jnp.maximum(ea_ref[...] @ W1c_ref[...] + b1c_ref[...][None, :], 0.0)
    z = h @ W2c_ref[...] + b2c_ref[...][None, :]
    w12 = jax.nn.sigmoid(z)
    gid = i * BE + lax.broadcasted_iota(jnp.int32, (BE,), 0)
    valid = gid < E
    w1_ref[...] = jnp.where(valid, w12[:, 0], 0.0)
    w2_ref[...] = jnp.where(valid, w12[:, 1], 0.0)


def _edge_weights(ea_p, mW1_1, mb1_1, mW2_1, mb2_1, mW1_2, mb1_2, mW2_2, mb2_2):
    W1c = jnp.concatenate([mW1_1, mW1_2], axis=1)
    b1c = jnp.concatenate([mb1_1, mb1_2])
    W2c = jnp.zeros((2 * H, 8), jnp.float32)
    W2c = W2c.at[:H, 0].set(mW2_1[:, 0]).at[H:, 1].set(mW2_2[:, 0])
    b2c = jnp.zeros((8,), jnp.float32)
    b2c = b2c.at[0].set(mb2_1[0]).at[1].set(mb2_2[0])
    grid = EPAD // BE
    return pl.pallas_call(
        _emlp_body,
        grid=(grid,),
        in_specs=[
            pl.BlockSpec((BE, DE), lambda i: (i, 0)),
            pl.BlockSpec((DE, 2 * H), lambda i: (0, 0)),
            pl.BlockSpec((2 * H,), lambda i: (0,)),
            pl.BlockSpec((2 * H, 8), lambda i: (0, 0)),
            pl.BlockSpec((8,), lambda i: (0,)),
        ],
        out_specs=[
            pl.BlockSpec((BE,), lambda i: (i,)),
            pl.BlockSpec((BE,), lambda i: (i,)),
        ],
        out_shape=[
            jax.ShapeDtypeStruct((EPAD,), jnp.float32),
            jax.ShapeDtypeStruct((EPAD,), jnp.float32),
        ],
    )(ea_p, W1c, b1c, W2c, b2c)


def _y1_body(x_ref, linT_ref, deg_ref, y_ref):
    dis = lax.rsqrt(deg_ref[...])
    y_ref[0] = dis * (x_ref[...] @ linT_ref[0])


def _y1(x, lin1, deg1):
    linT = jnp.transpose(lin1.reshape(D, 2, D), (1, 0, 2))
    return pl.pallas_call(
        _y1_body,
        grid=(2, N // BN),
        in_specs=[
            pl.BlockSpec((BN, D), lambda j, i: (i, 0)),
            pl.BlockSpec((1, D, D), lambda j, i: (j, 0, 0)),
            pl.BlockSpec((BN, 1), lambda j, i: (i, 0)),
        ],
        out_specs=pl.BlockSpec((1, BN, D), lambda j, i: (j, i, 0)),
        out_shape=jax.ShapeDtypeStruct((2, N, D), jnp.float32),
    )(x, linT, deg1)


def _y2_body(acc1_ref, deg1_ref, b1_ref, lin2T_ref, deg2_ref, y2_ref):
    dis1 = lax.rsqrt(deg1_ref[...])
    x1h0 = _leaky(dis1 * acc1_ref[0] + b1_ref[0][None, :])
    x1h1 = _leaky(dis1 * acc1_ref[1] + b1_ref[1][None, :])
    xl2 = x1h0 @ lin2T_ref[0] + x1h1 @ lin2T_ref[1]
    y2_ref[...] = lax.rsqrt(deg2_ref[...]) * xl2


def _y2(acc1, deg1, b1, lin2, deg2):
    lin2T = lin2.reshape(2, D, D)
    b1r = b1.reshape(2, D)
    return pl.pallas_call(
        _y2_body,
        grid=(N // BN,),
        in_specs=[
            pl.BlockSpec((2, BN, D), lambda i: (0, i, 0)),
            pl.BlockSpec((BN, 1), lambda i: (i, 0)),
            pl.BlockSpec((2, D), lambda i: (0, 0)),
            pl.BlockSpec((2, D, D), lambda i: (0, 0, 0)),
            pl.BlockSpec((BN, 1), lambda i: (i, 0)),
        ],
        out_specs=pl.BlockSpec((BN, D), lambda i: (i, 0)),
        out_shape=jax.ShapeDtypeStruct((N, D), jnp.float32),
    )(acc1, deg1, b1r, lin2T, deg2)


def _final_body(s20_ref, s21_ref, y2_ref, x_ref, deg2_ref, b2_ref, fcW_ref,
                fcb_ref, out_ref):
    i = pl.program_id(0)
    dis2 = lax.rsqrt(deg2_ref[...])
    x2 = (dis2 * (s20_ref[...] + s21_ref[...] - y2_ref[...])
          + b2_ref[...][None, :] + x_ref[...])
    m = jnp.max(x2, axis=0, keepdims=True)

    @pl.when(i == 0)
    def _():
        out_ref[...] = m

    @pl.when(i > 0)
    def _():
        out_ref[...] = jnp.maximum(out_ref[...], m)

    @pl.when(i == pl.num_programs(0) - 1)
    def _():
        g = _leaky(out_ref[...])
        out_ref[...] = g @ fcW_ref[...] + fcb_ref[...][None, :]


def _final(s20, s21, y2, x, deg2, b2, fcW, fcb):
    return pl.pallas_call(
        _final_body,
        grid=(N // BN,),
        in_specs=[
            pl.BlockSpec((BN, D), lambda i: (i, 0)),
            pl.BlockSpec((BN, D), lambda i: (i, 0)),
            pl.BlockSpec((BN, D), lambda i: (i, 0)),
            pl.BlockSpec((BN, D), lambda i: (i, 0)),
            pl.BlockSpec((BN, 1), lambda i: (i, 0)),
            pl.BlockSpec((D,), lambda i: (0,)),
            pl.BlockSpec((D, D), lambda i: (0, 0)),
            pl.BlockSpec((D,), lambda i: (0,)),
        ],
        out_specs=pl.BlockSpec((1, D), lambda i: (0, 0)),
        out_shape=jax.ShapeDtypeStruct((1, D), jnp.float32),
    )(s20, s21, y2, x, deg2, b2, fcW, fcb)


def kernel(x, edge_index, edge_attr, batch, lin1, mW1_1, mb1_1, mW2_1, mb2_1,
           b1, lin2, mW1_2, mb1_2, mW2_2, mb2_2, b2, fcW, fcb):
    pad = EPAD - E
    ea_p = jnp.pad(edge_attr, ((0, pad), (0, 0)))
    row_p = jnp.pad(edge_index[0], (0, pad))
    col_p = jnp.pad(edge_index[1], (0, pad))

    w1_p, w2_p = _edge_weights(ea_p, mW1_1, mb1_1, mW2_1, mb2_1,
                               mW1_2, mb1_2, mW2_2, mb2_2)

    deg1 = (jax.ops.segment_sum(w1_p, col_p, num_segments=N) + 1.0)[:, None]
    deg2 = (jax.ops.segment_sum(w2_p, col_p, num_segments=N) + 1.0)[:, None]

    y1 = _y1(x, lin1, deg1)
    y1cat = jnp.concatenate([y1[0], y1[1]], axis=1)
    s1 = jax.ops.segment_sum(w1_p[:, None] * y1cat[row_p], col_p,
                             num_segments=N)
    acc1 = y1 + jnp.stack([s1[:, :D], s1[:, D:]])

    y2 = _y2(acc1, deg1, b1, lin2, deg2)
    s2 = jax.ops.segment_sum(w2_p[:, None] * y2[row_p], col_p, num_segments=N)

    return _final(s2 + y2, y2, y2, x, deg2, b2, fcW, fcb)


# speedup vs baseline: 5.1091x; 2.2740x over previous
import functools

import jax
import jax.numpy as jnp
from jax import lax
from jax.experimental import pallas as pl
from jax.experimental.pallas import tpu as pltpu
from jax.experimental.pallas import tpu_sc as plsc

N, E, D, DE, H = 10000, 320000, 128, 16, 64
EPAD = 323584
BE = 4096
BN = 1000
F = 64

NT = 16
CHUNK = 128
PT1 = EPAD // NT
NC1 = PT1 // CHUNK
PT2 = EPAD // (2 * NT)
NC2 = PT2 // CHUNK
NP = 10240
NSL = NP // NT


def _sc_mesh():
    return plsc.VectorSubcoreMesh(core_axis_name="c", subcore_axis_name="s")


def _leaky(v):
    return jnp.where(v >= 0, v, 0.01 * v)


def _emlp_body(ea_ref, W1c_ref, b1c_ref, W2c_ref, b2c_ref, w1_ref, w2_ref):
    i = pl.program_id(0)
    h = jnp.maximum(ea_ref[...] @ W1c_ref[...] + b1c_ref[...][None, :], 0.0)
    z = h @ W2c_ref[...] + b2c_ref[...][None, :]
    w12 = jax.nn.sigmoid(z)
    gid = i * BE + lax.broadcasted_iota(jnp.int32, (BE,), 0)
    valid = gid < E
    w1_ref[...] = jnp.where(valid, w12[:, 0], 0.0)
    w2_ref[...] = jnp.where(valid, w12[:, 1], 0.0)


def _edge_weights(ea_p, mW1_1, mb1_1, mW2_1, mb2_1, mW1_2, mb1_2, mW2_2, mb2_2):
    W1c = jnp.concatenate([mW1_1, mW1_2], axis=1)
    b1c = jnp.concatenate([mb1_1, mb1_2])
    W2c = jnp.zeros((2 * H, 8), jnp.float32)
    W2c = W2c.at[:H, 0].set(mW2_1[:, 0]).at[H:, 1].set(mW2_2[:, 0])
    b2c = jnp.zeros((8,), jnp.float32)
    b2c = b2c.at[0].set(mb2_1[0]).at[1].set(mb2_2[0])
    return pl.pallas_call(
        _emlp_body,
        grid=(EPAD // BE,),
        in_specs=[
            pl.BlockSpec((BE, DE), lambda i: (i, 0)),
            pl.BlockSpec((DE, 2 * H), lambda i: (0, 0)),
            pl.BlockSpec((2 * H,), lambda i: (0,)),
            pl.BlockSpec((2 * H, 8), lambda i: (0, 0)),
            pl.BlockSpec((8,), lambda i: (0,)),
        ],
        out_specs=[
            pl.BlockSpec((BE,), lambda i: (i,)),
            pl.BlockSpec((BE,), lambda i: (i,)),
        ],
        out_shape=[
            jax.ShapeDtypeStruct((EPAD,), jnp.float32),
            jax.ShapeDtypeStruct((EPAD,), jnp.float32),
        ],
    )(ea_p, W1c, b1c, W2c, b2c)


def _y1_body(x_ref, linT_ref, deg_ref, y_ref):
    dis = lax.rsqrt(deg_ref[...] + 1.0)
    y_ref[0] = dis * (x_ref[...] @ linT_ref[0])


def _y1(x, lin1, deg1):
    linT = jnp.transpose(lin1.reshape(D, 4, F), (1, 0, 2))
    return pl.pallas_call(
        _y1_body,
        grid=(4, N // BN),
        in_specs=[
            pl.BlockSpec((BN, D), lambda j, i: (i, 0)),
            pl.BlockSpec((1, D, F), lambda j, i: (j, 0, 0)),
            pl.BlockSpec((BN, 1), lambda j, i: (i, 0)),
        ],
        out_specs=pl.BlockSpec((1, BN, F), lambda j, i: (j, i, 0)),
        out_shape=jax.ShapeDtypeStruct((4, NP, F), jnp.float32),
    )(x, linT, deg1)


def _y2_body(acc1_ref, deg1_ref, b1_ref, lin2T_ref, deg2_ref, y2_ref):
    dis1 = lax.rsqrt(deg1_ref[...] + 1.0)
    xl2 = jnp.zeros((BN, D), jnp.float32)
    for q in range(4):
        x1q = _leaky(dis1 * acc1_ref[q] + b1_ref[q][None, :])
        xl2 = xl2 + x1q @ lin2T_ref[q]
    dis2 = lax.rsqrt(deg2_ref[...] + 1.0)
    y2_ref[0] = dis2 * xl2[:, :F]
    y2_ref[1] = dis2 * xl2[:, F:]


def _y2(acc1, deg1, b1, lin2, deg2):
    lin2T = lin2.reshape(4, F, D)
    b1r = b1.reshape(4, F)
    return pl.pallas_call(
        _y2_body,
        grid=(N // BN,),
        in_specs=[
            pl.BlockSpec((4, BN, F), lambda i: (0, i, 0)),
            pl.BlockSpec((BN, 1), lambda i: (i, 0)),
            pl.BlockSpec((4, F), lambda i: (0, 0)),
            pl.BlockSpec((4, F, D), lambda i: (0, 0, 0)),
            pl.BlockSpec((BN, 1), lambda i: (i, 0)),
        ],
        out_specs=pl.BlockSpec((2, BN, F), lambda i: (0, i, 0)),
        out_shape=jax.ShapeDtypeStruct((2, NP, F), jnp.float32),
    )(acc1, deg1, b1r, lin2T, deg2)


def _final_body(s2_ref, y2_ref, x_ref, deg2_ref, b2_ref, fcW_ref,
                fcb_ref, out_ref):
    i = pl.program_id(0)
    dis2 = lax.rsqrt(deg2_ref[...] + 1.0)
    ms = []
    for h in range(2):
        x2h = (dis2 * (s2_ref[0, h] + s2_ref[1, h] - y2_ref[h])
               + b2_ref[h][None, :] + x_ref[:, h * F:(h + 1) * F])
        ms.append(jnp.max(x2h, axis=0, keepdims=True))
    m = jnp.concatenate(ms, axis=1)

    @pl.when(i == 0)
    def _():
        out_ref[...] = m

    @pl.when(i > 0)
    def _():
        out_ref[...] = jnp.maximum(out_ref[...], m)

    @pl.when(i == pl.num_programs(0) - 1)
    def _():
        g = _leaky(out_ref[...])
        out_ref[...] = g @ fcW_ref[...] + fcb_ref[...][None, :]


def _final(s2, y2, x, deg2, b2, fcW, fcb):
    b2r = b2.reshape(2, F)
    return pl.pallas_call(
        _final_body,
        grid=(N // BN,),
        in_specs=[
            pl.BlockSpec((2, 2, BN, F), lambda i: (0, 0, i, 0)),
            pl.BlockSpec((2, BN, F), lambda i: (0, i, 0)),
            pl.BlockSpec((BN, D), lambda i: (i, 0)),
            pl.BlockSpec((BN, 1), lambda i: (i, 0)),
            pl.BlockSpec((2, F), lambda i: (0, 0)),
            pl.BlockSpec((D, D), lambda i: (0, 0)),
            pl.BlockSpec((D,), lambda i: (0,)),
        ],
        out_specs=pl.BlockSpec((1, D), lambda i: (0, 0)),
        out_shape=jax.ShapeDtypeStruct((1, D), jnp.float32),
    )(s2, y2, x, deg2, b2r, fcW, fcb)


def _sc_deg(col_p, w1_p, w2_p):
    @functools.partial(
        pl.kernel,
        out_type=[jax.ShapeDtypeStruct((NP,), jnp.float32),
                  jax.ShapeDtypeStruct((NP,), jnp.float32)],
        mesh=_sc_mesh(),
        compiler_params=pltpu.CompilerParams(use_tc_tiling_on_sc=False),
        scratch_types=[
            pltpu.VMEM((CHUNK,), jnp.int32),
            pltpu.VMEM((CHUNK,), jnp.float32),
            pltpu.VMEM((NP,), jnp.float32),
            pltpu.VMEM_SHARED((NP,), jnp.float32),
        ],
    )
    def deg_kernel(col_hbm, w1_hbm, w2_hbm, d1_hbm, d2_hbm,
                   col_v, w_v, z_v, acc):
        c = lax.axis_index("c")
        s = lax.axis_index("s")

        @pl.when(s == 0)
        def _():
            def zb(i, carry):
                z_v[pl.ds(i * 16, 16)] = jnp.zeros((16,), jnp.float32)
                return carry
            lax.fori_loop(0, NP // 16, zb, 0)
            pltpu.sync_copy(z_v, acc)

        plsc.subcore_barrier()

        def run(w_hbm):
            base = s * PT1

            def body(k, carry):
                off = base + k * CHUNK
                pltpu.sync_copy(col_hbm.at[pl.ds(off, CHUNK)], col_v)
                pltpu.sync_copy(w_hbm.at[pl.ds(off, CHUNK)], w_v)
                pltpu.sync_copy(w_v, acc.at[col_v], add=True)
                return carry
            lax.fori_loop(0, NC1, body, 0)

        @pl.when(c == 0)
        def _():
            run(w1_hbm)

        @pl.when(c == 1)
        def _():
            run(w2_hbm)

        plsc.subcore_barrier()

        @pl.when(s == 0)
        def _():
            @pl.when(c == 0)
            def _():
                pltpu.sync_copy(acc, d1_hbm)

            @pl.when(c == 1)
            def _():
                pltpu.sync_copy(acc, d2_hbm)

    return deg_kernel(col_p, w1_p, w2_p)


def _agg_pass(y_grp, s_grp, row_hbm, col_hbm, w_hbm,
              row_v, col_v, w_v, rows_v, stage, acc, sem,
              tile, edge_base, n_chunks):
    nb = tile * NSL
    pltpu.sync_copy(y_grp.at[pl.ds(nb, NSL)], stage)
    pltpu.sync_copy(stage, acc.at[pl.ds(nb, NSL)])
    plsc.subcore_barrier()

    def body(k, carry):
        off = edge_base + k * CHUNK
        pltpu.sync_copy(row_hbm.at[pl.ds(off, CHUNK)], row_v)
        pltpu.sync_copy(col_hbm.at[pl.ds(off, CHUNK)], col_v)
        pltpu.sync_copy(w_hbm.at[pl.ds(off, CHUNK)], w_v)
        pltpu.async_copy(y_grp.at[row_v], rows_v, sem).wait()

        def scale(g, carry2):
            e0 = g * 16
            w16 = w_v[pl.ds(e0, 16)]
            for j in range(16):
                ws = jnp.broadcast_to(w16[j], (16,))
                for f in range(F // 16):
                    sl = pl.ds(f * 16, 16)
                    rows_v[e0 + j, sl] = rows_v[e0 + j, sl] * ws
            return carry2
        lax.fori_loop(0, CHUNK // 16, scale, 0)

        pltpu.sync_copy(rows_v, acc.at[col_v], add=True)
        return carry
    lax.fori_loop(0, n_chunks, body, 0)
    plsc.subcore_barrier()
    pltpu.sync_copy(acc.at[pl.ds(nb, NSL)], stage)
    pltpu.sync_copy(stage, s_grp.at[pl.ds(nb, NSL)])


def _agg_scratch():
    return [
        pltpu.VMEM((CHUNK,), jnp.int32),
        pltpu.VMEM((CHUNK,), jnp.int32),
        pltpu.VMEM((CHUNK,), jnp.float32),
        pltpu.VMEM((CHUNK, F), jnp.float32),
        pltpu.VMEM((NSL, F), jnp.float32),
        pltpu.VMEM_SHARED((NP, F), jnp.float32),
        pltpu.SemaphoreType.DMA,
    ]


def _sc_agg1(row_p, col_p, w1_p, y1):
    @functools.partial(
        pl.kernel,
        out_type=jax.ShapeDtypeStruct((4, NP, F), jnp.float32),
        mesh=_sc_mesh(),
        compiler_params=pltpu.CompilerParams(use_tc_tiling_on_sc=False),
        scratch_types=_agg_scratch(),
    )
    def agg1(row_hbm, col_hbm, w_hbm, y_hbm, s_hbm,
             row_v, col_v, w_v, rows_v, stage, acc, sem):
        c = lax.axis_index("c")
        s = lax.axis_index("s")
        for qi in range(2):
            q = 2 * c + qi
            plsc.subcore_barrier()
            _agg_pass(y_hbm.at[q], s_hbm.at[q], row_hbm, col_hbm, w_hbm,
                      row_v, col_v, w_v, rows_v, stage, acc, sem,
                      s, s * PT1, NC1)

    return agg1(row_p, col_p, w1_p, y1)


def _sc_agg2(row_p, col_p, w2_p, y2):
    @functools.partial(
        pl.kernel,
        out_type=jax.ShapeDtypeStruct((2, 2, NP, F), jnp.float32),
        mesh=_sc_mesh(),
        compiler_params=pltpu.CompilerParams(use_tc_tiling_on_sc=False),
        scratch_types=_agg_scratch(),
    )
    def agg2(row_hbm, col_hbm, w_hbm, y_hbm, s_hbm,
             row_v, col_v, w_v, rows_v, stage, acc, sem):
        c = lax.axis_index("c")
        s = lax.axis_index("s")
        for h in range(2):
            plsc.subcore_barrier()
            _agg_pass(y_hbm.at[h], s_hbm.at[c].at[h], row_hbm, col_hbm, w_hbm,
                      row_v, col_v, w_v, rows_v, stage, acc, sem,
                      s, c * (EPAD // 2) + s * PT2, NC2)

    return agg2(row_p, col_p, w2_p, y2)


def kernel(x, edge_index, edge_attr, batch, lin1, mW1_1, mb1_1, mW2_1, mb2_1,
           b1, lin2, mW1_2, mb1_2, mW2_2, mb2_2, b2, fcW, fcb):
    pad = EPAD - E
    ea_p = jnp.pad(edge_attr, ((0, pad), (0, 0)))
    row_p = jnp.pad(edge_index[0], (0, pad))
    col_p = jnp.pad(edge_index[1], (0, pad))

    w1_p, w2_p = _edge_weights(ea_p, mW1_1, mb1_1, mW2_1, mb2_1,
                               mW1_2, mb1_2, mW2_2, mb2_2)

    d1, d2 = _sc_deg(col_p, w1_p, w2_p)
    deg1, deg2 = d1[:, None], d2[:, None]

    y1 = _y1(x, lin1, deg1)
    acc1 = _sc_agg1(row_p, col_p, w1_p, y1)
    y2 = _y2(acc1, deg1, b1, lin2, deg2)
    s2 = _sc_agg2(row_p, col_p, w2_p, y2)

    return _final(s2, y2, x, deg2, b2, fcW, fcb)
